# Initial kernel scaffold; baseline (speedup 1.0000x reference)
import jax
import jax.numpy as jnp
from jax.experimental import pallas as pl


def kernel(x, edge_index, batch, params):
    raise NotImplementedError("write your pallas kernel here")


# speedup vs baseline: 6.1772x; 6.1772x over previous
import functools
import math

import jax
import jax.numpy as jnp
from jax import lax
from jax.experimental import pallas as pl
from jax.experimental.pallas import tpu as pltpu
from jax.experimental.pallas import tpu_sc as plsc

_N = 10000
_E = 320000
_D = 128
_NG = 64
_BN_EPS = 1e-5
_SLOPE = 0.01

_CHUNK = 128
_NCHUNKS = _E // _CHUNK
_ROWS_A = 624


def _agg_body(h_hbm, src_hbm, dst_hbm, out_hbm, idx_s, idx_d, buf, zbuf, acc,
              sem):
    cid = lax.axis_index("c")
    sid = lax.axis_index("s")

    def zstore(i, carry):
        r = i // 8
        c = (i % 8) * 16
        zbuf[r, pl.ds(c, 16)] = jnp.zeros((16,), jnp.float32)
        return carry

    lax.fori_loop(0, 128, zstore, 0)

    row0 = sid * _ROWS_A
    nz = jnp.where(sid == 15, 40, 39)

    def zcopy(j, carry):
        pltpu.sync_copy(zbuf, acc.at[pl.ds(row0 + j * 16, 16)])
        return carry

    lax.fori_loop(0, nz, zcopy, 0)
    plsc.subcore_barrier()

    wid = cid * 16 + sid
    nc = (_NCHUNKS - wid + 31) // 32

    def echunk(g, carry):
        ci = wid + g * 32
        pltpu.sync_copy(src_hbm.at[ci], idx_s)
        pltpu.sync_copy(dst_hbm.at[ci], idx_d)
        pltpu.async_copy(h_hbm.at[idx_s], buf, sem).wait()
        pltpu.sync_copy(buf, acc.at[idx_d], add=True)
        return carry

    lax.fori_loop(0, nc, echunk, 0)
    plsc.subcore_barrier()

    @pl.when(sid != 15)
    def _():
        pltpu.sync_copy(
            acc.at[pl.ds(row0, _ROWS_A)],
            out_hbm.at[pl.ds(cid * _N + row0, _ROWS_A)],
        )

    @pl.when(sid == 15)
    def _():
        pltpu.sync_copy(
            acc.at[pl.ds(row0, _N - 15 * _ROWS_A)],
            out_hbm.at[pl.ds(cid * _N + row0, _N - 15 * _ROWS_A)],
        )


@jax.jit
def _edge_agg(h, src_c, dst_c):
    mesh = plsc.VectorSubcoreMesh(core_axis_name="c", subcore_axis_name="s")
    fn = pl.kernel(
        _agg_body,
        mesh=mesh,
        out_type=jax.ShapeDtypeStruct((2 * _N, _D), jnp.float32),
        scratch_types=[
            pltpu.VMEM((_CHUNK,), jnp.int32),
            pltpu.VMEM((_CHUNK,), jnp.int32),
            pltpu.VMEM((_CHUNK, _D), jnp.float32),
            pltpu.VMEM((16, _D), jnp.float32),
            pltpu.VMEM_SHARED((_N, _D), jnp.float32),
            pltpu.SemaphoreType.DMA,
        ],
    )
    return fn(h, src_c, dst_c)


_BNF = 1.0 / math.sqrt(1.0 + _BN_EPS)


def _conv_tc_body(h_ref, agg_ref, w_ref, b_ref, g_ref, bt_ref, ep_ref, o_ref):
    a = agg_ref[0:_N, :] + agg_ref[_N:2 * _N, :]
    x2 = (1.0 + ep_ref[...]) * h_ref[...] + a
    t = jnp.dot(x2, w_ref[...], preferred_element_type=jnp.float32)
    t = (t + b_ref[...]) * (g_ref[...] * _BNF) + bt_ref[...]
    o_ref[...] = jnp.where(t >= 0.0, t, t * (_SLOPE * _SLOPE))


@jax.jit
def _conv_update(h, agg2, w, b, gamma, beta, epsv):
    return pl.pallas_call(
        _conv_tc_body,
        out_shape=jax.ShapeDtypeStruct((_N, _D), jnp.float32),
    )(h, agg2, w, b, gamma, beta, epsv)


def _head_body(g2_ref, g3_ref, g4_ref, bat_ref, w1_ref, b1_ref, w2_ref,
               b2_ref, w3_ref, b3_ref, wf_ref, bf_ref, o_ref):
    bat = bat_ref[...]
    gi = lax.broadcasted_iota(jnp.int32, (_NG, _N), 0)
    oh = jnp.where(gi == bat, 1.0, 0.0).astype(jnp.float32)
    g4 = g4_ref[...]
    pool = jnp.dot(oh, g4, preferred_element_type=jnp.float32)
    hp = lax.dot_general(oh, pool, (((0,), (0,)), ((), ())),
                         preferred_element_type=jnp.float32)
    w1 = w1_ref[...]
    z = jnp.dot(g2_ref[...], w1[0:_D, :], preferred_element_type=jnp.float32)
    z = z + jnp.dot(g3_ref[...], w1[_D:2 * _D, :],
                    preferred_element_type=jnp.float32)
    z = z + jnp.dot(g4, w1[2 * _D:3 * _D, :],
                    preferred_element_type=jnp.float32)
    z = z + jnp.dot(hp, w1[3 * _D:4 * _D, :],
                    preferred_element_type=jnp.float32)
    z = z + b1_ref[...]
    z = jnp.dot(z, w2_ref[...], preferred_element_type=jnp.float32) + b2_ref[...]
    z = jnp.where(z >= 0.0, z, z * _SLOPE)
    z = jnp.dot(z, w3_ref[...], preferred_element_type=jnp.float32) + b3_ref[...]
    z = jnp.where(z >= 0.0, z, z * _SLOPE)
    z = jnp.dot(z, wf_ref[...], preferred_element_type=jnp.float32) + bf_ref[...]
    o_ref[...] = 1.0 / (1.0 + jnp.exp(-z))


@jax.jit
def _head(g2, g3, g4, batf, w1, b1, w2, b2, w3, b3, wfp, bfp):
    return pl.pallas_call(
        _head_body,
        out_shape=jax.ShapeDtypeStruct((_N, _D), jnp.float32),
    )(g2, g3, g4, batf, w1, b1, w2, b2, w3, b3, wfp, bfp)


def kernel(x, edge_index, batch, params):
    src_c = edge_index[0].astype(jnp.int32).reshape(_NCHUNKS, _CHUNK)
    dst_c = edge_index[1].astype(jnp.int32).reshape(_NCHUNKS, _CHUNK)
    batf = batch.astype(jnp.int32).reshape(1, _N)

    def conv_params(p):
        return (p['W'], p['b'].reshape(1, _D), p['gamma'].reshape(1, _D),
                p['beta'].reshape(1, _D),
                jnp.broadcast_to(p['eps'].reshape(1, 1), (1, _D)))

    h = x
    hs = []
    for i, p in enumerate([params['conv1']] + list(params['convs'])):
        agg2 = _edge_agg(h, src_c, dst_c)
        w, b, gamma, beta, epsv = conv_params(p)
        h = _conv_update(h, agg2, w, b, gamma, beta, epsv)
        if i > 0:
            hs.append(h)

    wfp = jnp.pad(params['final']['W'], ((0, 0), (0, _D - 1)))
    bfp = jnp.pad(params['final']['b'], (0, _D - 1)).reshape(1, _D)
    out = _head(
        hs[0], hs[1], hs[2], batf,
        params['cls1']['W'], params['cls1']['b'].reshape(1, _D),
        params['cls'][0]['W'], params['cls'][0]['b'].reshape(1, _D),
        params['cls'][1]['W'], params['cls'][1]['b'].reshape(1, _D),
        wfp, bfp,
    )
    return out[:, :1]
